# baseline (speedup 1.0000x reference)
import jax
import jax.numpy as jnp
from jax.experimental import pallas as pl


def _add_kernel(x_ref, w_ref, o_ref):
    o_ref[...] = x_ref[...] + w_ref[...]


def kernel(inputs, embed_weight):
    bs, T, D = inputs.shape
    blk = 256
    return pl.pallas_call(
        _add_kernel,
        grid=(T // blk, bs),
        in_specs=[
            pl.BlockSpec((1, blk, D), lambda t, b: (b, t, 0)),
            pl.BlockSpec((blk, D), lambda t, b: (t, 0)),
        ],
        out_specs=pl.BlockSpec((1, blk, D), lambda t, b: (b, t, 0)),
        out_shape=jax.ShapeDtypeStruct(inputs.shape, inputs.dtype),
    )(inputs, embed_weight)


# speedup vs baseline: 1.3045x; 1.3045x over previous
import jax
import jax.numpy as jnp
from jax.experimental import pallas as pl


def _add_kernel(x_ref, w_ref, o_ref):
    o_ref[...] = x_ref[...] + w_ref[...]


def kernel(inputs, embed_weight):
    bs, T, D = inputs.shape
    blk = 512
    nt = T // blk
    x2 = inputs.reshape(bs * T, D)
    out = pl.pallas_call(
        _add_kernel,
        grid=(nt, bs),
        in_specs=[
            pl.BlockSpec((blk, D), lambda t, b: (b * nt + t, 0)),
            pl.BlockSpec((blk, D), lambda t, b: (t, 0)),
        ],
        out_specs=pl.BlockSpec((blk, D), lambda t, b: (b * nt + t, 0)),
        out_shape=jax.ShapeDtypeStruct((bs * T, D), inputs.dtype),
    )(x2, embed_weight)
    return out.reshape(bs, T, D)


# speedup vs baseline: 1.4250x; 1.0924x over previous
import jax
import jax.numpy as jnp
from jax.experimental import pallas as pl


def _add_kernel(x_ref, w_ref, o_ref):
    o_ref[...] = x_ref[...] + w_ref[...]


def kernel(inputs, embed_weight):
    bs, T, D = inputs.shape
    blk = 1024
    nt = T // blk
    x2 = inputs.reshape(bs * T, D)
    out = pl.pallas_call(
        _add_kernel,
        grid=(nt, bs),
        in_specs=[
            pl.BlockSpec((blk, D), lambda t, b: (b * nt + t, 0)),
            pl.BlockSpec((blk, D), lambda t, b: (t, 0)),
        ],
        out_specs=pl.BlockSpec((blk, D), lambda t, b: (b * nt + t, 0)),
        out_shape=jax.ShapeDtypeStruct((bs * T, D), inputs.dtype),
    )(x2, embed_weight)
    return out.reshape(bs, T, D)


# speedup vs baseline: 1.5424x; 1.0823x over previous
import jax
import jax.numpy as jnp
from jax.experimental import pallas as pl


def _add_kernel(x_ref, w_ref, o_ref):
    o_ref[...] = x_ref[...] + w_ref[...]


def kernel(inputs, embed_weight):
    bs, T, D = inputs.shape
    blk = 2048
    nt = T // blk
    x2 = inputs.reshape(bs * T, D)
    out = pl.pallas_call(
        _add_kernel,
        grid=(nt, bs),
        in_specs=[
            pl.BlockSpec((blk, D), lambda t, b: (b * nt + t, 0)),
            pl.BlockSpec((blk, D), lambda t, b: (t, 0)),
        ],
        out_specs=pl.BlockSpec((blk, D), lambda t, b: (b * nt + t, 0)),
        out_shape=jax.ShapeDtypeStruct((bs * T, D), inputs.dtype),
    )(x2, embed_weight)
    return out.reshape(bs, T, D)
